# baseline (speedup 1.0000x reference)
import functools

import jax
import jax.numpy as jnp
from jax import lax
from jax.experimental import pallas as pl
from jax.experimental.pallas import tpu as pltpu
from jax.experimental.pallas import tpu_sc as plsc

BATCH = 4096
SEQ = 200
D = 64
NC = 2
NS = 16
NW = NC * NS
SEQ_PER_W = BATCH // NW
SCALE = 8.0

SPLITS = ((0, 104), (104, 96))


def _body(idx_hbm, tok_hbm, pos_hbm, out_hbm, idx_v, pos_v, rows_v, out_v, sem):
    wid = lax.axis_index("s") * NC + lax.axis_index("c")

    pltpu.sync_copy(pos_hbm, pos_v)

    def seq_body(i, carry):
        base = (wid * SEQ_PER_W + i) * SEQ
        pltpu.sync_copy(idx_hbm.at[pl.ds(base, SEQ)], idx_v)
        for off, ln in SPLITS:
            pltpu.async_copy(
                tok_hbm.at[idx_v.at[pl.ds(off, ln)]],
                rows_v.at[pl.ds(off, ln)],
                sem,
            )
        for off, ln in SPLITS:
            pltpu.make_async_copy(
                tok_hbm.at[idx_v.at[pl.ds(off, ln)]],
                rows_v.at[pl.ds(off, ln)],
                sem,
            ).wait()

        def row_body(j, c):
            for v in range(D // 16):
                sl = pl.ds(v * 16, 16)
                out_v[j, sl] = rows_v[j, sl] * SCALE + pos_v[j, sl]
            return c

        lax.fori_loop(0, SEQ, row_body, 0, unroll=2)
        pltpu.sync_copy(out_v, out_hbm.at[pl.ds(base, SEQ)])
        return carry

    lax.fori_loop(0, SEQ_PER_W, seq_body, 0)


@jax.jit
def _run(idx_flat, tok_table, pos_table):
    mesh = plsc.VectorSubcoreMesh(
        core_axis_name="c", subcore_axis_name="s", num_cores=NC, num_subcores=NS
    )
    k = pl.kernel(
        _body,
        out_type=jax.ShapeDtypeStruct((BATCH * SEQ, D), jnp.float32),
        mesh=mesh,
        scratch_types=[
            pltpu.VMEM((SEQ,), jnp.int32),
            pltpu.VMEM((SEQ, D), jnp.float32),
            pltpu.VMEM((SEQ, D), jnp.float32),
            pltpu.VMEM((SEQ, D), jnp.float32),
            pltpu.SemaphoreType.DMA,
        ],
        compiler_params=pltpu.CompilerParams(use_tc_tiling_on_sc=False),
    )
    return k(idx_flat, tok_table, pos_table)


def kernel(inputs, tok_table, pos_table):
    idx_flat = inputs.reshape(-1).astype(jnp.int32)
    out = _run(idx_flat, tok_table, pos_table)
    return out.reshape(BATCH, SEQ, D)


# speedup vs baseline: 1.5888x; 1.5888x over previous
import jax
import jax.numpy as jnp
from jax import lax
from jax.experimental import pallas as pl
from jax.experimental.pallas import tpu as pltpu
from jax.experimental.pallas import tpu_sc as plsc

BATCH = 4096
SEQ = 200
D = 64
NC = 2
NS = 16
NW = NC * NS
SEQ_PER_W = BATCH // NW
SCALE = 8.0
NBUF = 4

SPLITS = ((0, 104), (104, 96))


def _body(idx_hbm, tok_hbm, pos_hbm, out_hbm, idx_v, pos_v, bufs, gsems, osems):
    wid = lax.axis_index("s") * NC + lax.axis_index("c")

    pltpu.sync_copy(idx_hbm.at[pl.ds(wid * SEQ_PER_W * SEQ, SEQ_PER_W * SEQ)], idx_v)
    pltpu.sync_copy(pos_hbm, pos_v)

    def fire_gather(c, p):
        off = pl.multiple_of(c * SEQ, 8)
        for o, ln in SPLITS:
            pltpu.async_copy(
                tok_hbm.at[idx_v.at[pl.ds(off + o, ln)]],
                bufs[p].at[pl.ds(o, ln)],
                gsems[p],
            )

    def wait_gather(c, p):
        off = pl.multiple_of(c * SEQ, 8)
        for o, ln in SPLITS:
            pltpu.make_async_copy(
                tok_hbm.at[idx_v.at[pl.ds(off + o, ln)]],
                bufs[p].at[pl.ds(o, ln)],
                gsems[p],
            ).wait()

    def out_copy(c, p):
        base = pl.multiple_of((wid * SEQ_PER_W + c) * SEQ, 8)
        return pltpu.make_async_copy(
            bufs[p], out_hbm.at[pl.ds(base, SEQ)], osems[p]
        )

    fire_gather(0, 0)
    fire_gather(1, 1)

    def halfstep(t, p):
        c = t * NBUF + p
        pn = (p + 2) % NBUF

        def prefetch():
            out_copy(c - 2, pn).wait()
            fire_gather(c + 2, pn)

        if p >= 2:
            pl.when(t < 31)(prefetch)
        else:
            def prefetch_first():
                fire_gather(c + 2, pn)

            pl.when(t > 0)(prefetch)
            pl.when(t == 0)(prefetch_first)

        wait_gather(c, p)

        buf = bufs[p]

        @plsc.parallel_loop(0, SEQ, unroll=4)
        def row_body(j):
            for v in range(D // 16):
                sl = pl.ds(v * 16, 16)
                buf[j, sl] = buf[j, sl] * SCALE + pos_v[j, sl]

        out_copy(c, p).start()

    def body(t, carry):
        for p in range(NBUF):
            halfstep(t, p)
        return carry

    lax.fori_loop(0, SEQ_PER_W // NBUF, body, 0)

    for p in range(NBUF):
        c = SEQ_PER_W - NBUF + p
        out_copy(c, p).wait()


@jax.jit
def _run(idx_flat, tok_table, pos_table):
    mesh = plsc.VectorSubcoreMesh(
        core_axis_name="c", subcore_axis_name="s", num_cores=NC, num_subcores=NS
    )
    k = pl.kernel(
        _body,
        out_type=jax.ShapeDtypeStruct((BATCH * SEQ, D), jnp.float32),
        mesh=mesh,
        scratch_types=[
            pltpu.VMEM((SEQ_PER_W * SEQ,), jnp.int32),
            pltpu.VMEM((SEQ, D), jnp.float32),
            [pltpu.VMEM((SEQ, D), jnp.float32) for _ in range(NBUF)],
            [pltpu.SemaphoreType.DMA for _ in range(NBUF)],
            [pltpu.SemaphoreType.DMA for _ in range(NBUF)],
        ],
        compiler_params=pltpu.CompilerParams(use_tc_tiling_on_sc=False),
    )
    return k(idx_flat, tok_table, pos_table)


def kernel(inputs, tok_table, pos_table):
    idx_flat = inputs.reshape(-1).astype(jnp.int32)
    out = _run(idx_flat, tok_table, pos_table)
    return out.reshape(BATCH, SEQ, D)
